# baseline (speedup 1.0000x reference)
import functools

import jax
import jax.numpy as jnp
from jax import lax
from jax.experimental import pallas as pl
from jax.experimental.pallas import tpu as pltpu
from jax.experimental.pallas import tpu_sc as plsc

_CHUNK = 512
_NBUF = 4
_E = 16
_NEG = -3.0e38

_NC = 2
_NS = 16
_NW = _NC * _NS
_L = 16


def _matmul_body(x_hbm, w_ref, lt_ref, buf, sems):
    i = pl.program_id(0)
    n = pl.num_programs(0)

    @pl.when(i == 0)
    def _prime():
        for b in range(_NBUF):
            pltpu.make_async_copy(
                x_hbm.at[pl.ds(b * _CHUNK, _CHUNK), :],
                buf.at[b], sems.at[b]).start()

    slot = jax.lax.rem(i, _NBUF)
    pltpu.make_async_copy(
        x_hbm.at[pl.ds(i * _CHUNK, _CHUNK), :],
        buf.at[slot], sems.at[slot]).wait()

    logits = jax.lax.dot_general(
        buf[slot], w_ref[...],
        dimension_numbers=(((1,), (1,)), ((), ())),
        preferred_element_type=jnp.float32,
    )
    lt_ref[...] = logits.T

    @pl.when(i + _NBUF < n)
    def _prefetch():
        pltpu.make_async_copy(
            x_hbm.at[pl.ds((i + _NBUF) * _CHUNK, _CHUNK), :],
            buf.at[slot], sems.at[slot]).start()


@jax.jit
def _gate_logits_t(x2d, W):
    nt, d = x2d.shape
    return pl.pallas_call(
        _matmul_body,
        grid=(nt // _CHUNK,),
        in_specs=[
            pl.BlockSpec(memory_space=pl.ANY),
            pl.BlockSpec((_E, d), lambda i: (0, 0)),
        ],
        out_specs=pl.BlockSpec((_E, _CHUNK), lambda i: (0, i)),
        out_shape=jax.ShapeDtypeStruct((_E, nt), jnp.float32),
        scratch_shapes=[
            pltpu.VMEM((_NBUF, _CHUNK, d), jnp.float32),
            pltpu.SemaphoreType.DMA((_NBUF,)),
        ],
        compiler_params=pltpu.CompilerParams(
            dimension_semantics=("arbitrary",),
        ),
    )(x2d, W)


def _route_sc(lt):
    nt = lt.shape[1]
    tpw = nt // _NW

    mesh = plsc.VectorSubcoreMesh(core_axis_name="c", subcore_axis_name="s")

    @functools.partial(
        pl.kernel,
        out_type=[
            jax.ShapeDtypeStruct((2 * nt,), jnp.float32),
            jax.ShapeDtypeStruct((2 * nt,), jnp.int32),
        ],
        mesh=mesh,
        scratch_types=[
            pltpu.VMEM((_E, tpw), jnp.float32),
            pltpu.VMEM((2 * tpw,), jnp.float32),
            pltpu.VMEM((2 * tpw,), jnp.int32),
        ],
    )
    def sc_route(lt_hbm, wout_hbm, iout_hbm, lt_v, wp_v, ip_v):
        wid = lax.axis_index("s") * _NC + lax.axis_index("c")
        base = wid * tpw
        pltpu.sync_copy(lt_hbm.at[:, pl.ds(base, tpw)], lt_v)

        def group(t0):
            m1 = lt_v[0, pl.ds(t0, _L)]
            i1 = jnp.zeros((_L,), jnp.int32)
            m2 = jnp.full((_L,), _NEG, jnp.float32)
            i2 = jnp.zeros((_L,), jnp.int32)
            for e in range(1, _E):
                v = lt_v[e, pl.ds(t0, _L)]
                es = jnp.full((_L,), e, jnp.int32)
                gt1 = v > m1
                gt2 = v > m2
                new_m2 = jnp.where(gt1, m1, jnp.where(gt2, v, m2))
                new_i2 = jnp.where(gt1, i1, jnp.where(gt2, es, i2))
                m1 = jnp.where(gt1, v, m1)
                i1 = jnp.where(gt1, es, i1)
                m2 = new_m2
                i2 = new_i2
            z = jnp.exp(m2 - m1)
            w1 = 1.0 / (1.0 + z)
            w2 = z * w1
            wp_v[pl.ds(2 * t0, _L)] = w1
            wp_v[pl.ds(2 * t0 + _L, _L)] = w2
            ip_v[pl.ds(2 * t0, _L)] = i1
            ip_v[pl.ds(2 * t0 + _L, _L)] = i2

        for _g in range(tpw // _L):
            group(_g * _L)
        pltpu.sync_copy(wp_v, wout_hbm.at[pl.ds(2 * base, 2 * tpw)])
        pltpu.sync_copy(ip_v, iout_hbm.at[pl.ds(2 * base, 2 * tpw)])

    return sc_route(lt)


def _deinterleave(a, nt):
    return a.reshape(nt // _L, 2, _L).swapaxes(1, 2).reshape(nt, 2)


def kernel(x, W):
    B, T, D = x.shape
    nt = B * T
    lt = _gate_logits_t(x.reshape(nt, D), W)
    wflat, iflat = _route_sc(lt)
    w = _deinterleave(wflat, nt)
    i = _deinterleave(iflat, nt)
    return w.reshape(B, T, 2), i.reshape(B, T, 2)
